# baseline (speedup 1.0000x reference)
import jax
import jax.numpy as jnp
from jax.experimental import pallas as pl
from jax.experimental.pallas import tpu as pltpu

_B, _P, _NS = 16, 16384, 1024
_C = 128
_NCH = _P // _C


def _fps_body(pts_ref, idx_ref, sx_ref, sy_ref, sz_ref, dist_ref):
    dist_ref[...] = jnp.full((_B, _P), 1e10, jnp.float32)

    nxt0 = jnp.zeros((_B, 1), jnp.int32)
    cx0 = pts_ref[0, :, 0:1]
    cy0 = pts_ref[1, :, 0:1]
    cz0 = pts_ref[2, :, 0:1]

    def body(s, carry):
        nxt, cx, cy, cz = carry
        idx_ref[pl.ds(s, 1), :] = nxt.reshape(1, _B)
        sx_ref[pl.ds(s, 1), :] = cx.reshape(1, _B)
        sy_ref[pl.ds(s, 1), :] = cy.reshape(1, _B)
        sz_ref[pl.ds(s, 1), :] = cz.reshape(1, _B)

        macc = jnp.full((_B, _C), -1.0, jnp.float32)
        chacc = jnp.zeros((_B, _C), jnp.int32)
        xv = jnp.zeros((_B, _C), jnp.float32)
        yv = jnp.zeros((_B, _C), jnp.float32)
        zv = jnp.zeros((_B, _C), jnp.float32)
        for c in range(_NCH):
            o = c * _C
            px = pts_ref[0, :, o:o + _C]
            py = pts_ref[1, :, o:o + _C]
            pz = pts_ref[2, :, o:o + _C]
            dx = px - cx
            dy = py - cy
            dz = pz - cz
            d = (dx * dx + dz * dz) + dy * dy
            dd = jnp.minimum(dist_ref[:, o:o + _C], d)
            dist_ref[:, o:o + _C] = dd
            upd = dd > macc
            macc = jnp.where(upd, dd, macc)
            chacc = jnp.where(upd, c, chacc)
            xv = jnp.where(upd, px, xv)
            yv = jnp.where(upd, py, yv)
            zv = jnp.where(upd, pz, zv)

        base_iota = jax.lax.broadcasted_iota(jnp.int32, (_B, _C), 1)
        candidx = chacc * _C + base_iota
        m = jnp.max(macc, axis=1, keepdims=True)
        nxt2 = jnp.min(jnp.where(macc == m, candidx, _P),
                       axis=1, keepdims=True)
        sel = candidx == nxt2
        cx2 = jnp.max(jnp.where(sel, xv, -1e30), axis=1, keepdims=True)
        cy2 = jnp.max(jnp.where(sel, yv, -1e30), axis=1, keepdims=True)
        cz2 = jnp.max(jnp.where(sel, zv, -1e30), axis=1, keepdims=True)
        return nxt2, cx2, cy2, cz2

    jax.lax.fori_loop(0, _NS, body, (nxt0, cx0, cy0, cz0))


def _run(points, interpret=False):
    pts = jnp.transpose(points, (2, 0, 1))
    idx_t, sx, sy, sz = pl.pallas_call(
        _fps_body,
        out_shape=[
            jax.ShapeDtypeStruct((_NS, _B), jnp.int32),
            jax.ShapeDtypeStruct((_NS, _B), jnp.float32),
            jax.ShapeDtypeStruct((_NS, _B), jnp.float32),
            jax.ShapeDtypeStruct((_NS, _B), jnp.float32),
        ],
        scratch_shapes=[pltpu.VMEM((_B, _P), jnp.float32)],
        interpret=interpret,
    )(pts)
    indices = jnp.transpose(idx_t)
    sampled = jnp.stack([jnp.transpose(sx), jnp.transpose(sy),
                         jnp.transpose(sz)], axis=-1)
    return indices, sampled


def kernel(points, nsamples, kd_depth, return_gathered):
    return _run(points)
